# baseline (speedup 1.0000x reference)
import jax
import jax.numpy as jnp
from jax.experimental import pallas as pl

_B, _S, _D = 4, 4096, 2048
_N = _B * _S
_IDX = 3
_ROWS = 16384


def _tc_body(in_ref, out_ref):
    strip = in_ref[0, :, 0:8]
    out_ref[...] = jnp.swapaxes(strip, 0, 1)[_IDX]


def kernel(input_):
    x = input_.reshape(_N // _ROWS, _ROWS, _D)
    out = pl.pallas_call(
        _tc_body,
        grid=(_N // _ROWS,),
        in_specs=[pl.BlockSpec((1, _ROWS, 128), lambda i: (i, 0, 0))],
        out_specs=pl.BlockSpec((_ROWS,), lambda i: (i,)),
        out_shape=jax.ShapeDtypeStruct((_N,), jnp.float32),
    )(x)
    return out.reshape(_B, _S, 1)


# speedup vs baseline: 1.3439x; 1.3439x over previous
import jax
import jax.numpy as jnp
from jax.experimental import pallas as pl
from jax.experimental.pallas import tpu as pltpu

_B, _S, _D = 4, 4096, 2048
_N = _B * _S
_IDX = 3
_CHUNKS = 8
_ROWS = _N // _CHUNKS


def _body(in_hbm, out_ref, buf, sems):
    for k in range(_CHUNKS):
        pltpu.make_async_copy(
            in_hbm.at[k, :, pl.ds(0, 128)], buf.at[k], sems.at[k]
        ).start()
    for k in range(_CHUNKS):
        pltpu.make_async_copy(
            in_hbm.at[k, :, pl.ds(0, 128)], buf.at[k], sems.at[k]
        ).wait()
        strip = buf[k, :, 0:8]
        out_ref[pl.ds(k * _ROWS, _ROWS)] = jnp.swapaxes(strip, 0, 1)[_IDX]


def kernel(input_):
    x = input_.reshape(_CHUNKS, _ROWS, _D)
    out = pl.pallas_call(
        _body,
        in_specs=[pl.BlockSpec(memory_space=pl.ANY)],
        out_specs=pl.BlockSpec(memory_space=pltpu.VMEM),
        out_shape=jax.ShapeDtypeStruct((_N,), jnp.float32),
        scratch_shapes=[
            pltpu.VMEM((_CHUNKS, _ROWS, 128), jnp.float32),
            pltpu.SemaphoreType.DMA((_CHUNKS,)),
        ],
    )(x)
    return out.reshape(_B, _S, 1)
